# baseline (speedup 1.0000x reference)
import functools
import jax
import jax.numpy as jnp
from jax import lax
from jax.experimental import pallas as pl
from jax.experimental.pallas import tpu as pltpu
from jax.experimental.pallas import tpu_sc as plsc

B, L, H, K = 8, 2048, 768, 256
M = B * L
D = 384
TB = 512
NT = M // TB


def _dense_body(x_ref, w1_ref, b1_ref, w2_ref, b2_ref, wm_ref,
                out3_ref, pred_ref, ph_ref):
    x = x_ref[...]
    h1 = lax.dot_general(x, w1_ref[...], (((1,), (1,)), ((), ())),
                         preferred_element_type=jnp.float32)
    h1 = jnp.maximum(h1 + b1_ref[0:1, :], 0.0)
    logits8 = lax.dot_general(h1, w2_ref[...], (((1,), (1,)), ((), ())),
                              preferred_element_type=jnp.float32)
    logits8 = logits8 + b2_ref[0:1, :]
    lane8 = lax.broadcasted_iota(jnp.int32, (TB, 8), 1)
    neg = jnp.where(lane8 >= 3, -1e30, 0.0).astype(jnp.float32)
    lm = logits8 + neg
    m = jnp.max(lm, axis=1, keepdims=True)
    lse = m + jnp.log(jnp.sum(jnp.exp(lm - m), axis=1, keepdims=True))
    out8 = lm - lse
    out3_ref[...] = out8[:, 0:3]
    l0 = logits8[:, 0:1]
    l1 = logits8[:, 1:2]
    l2 = logits8[:, 2:3]
    pred = jnp.where(l0 >= l1,
                     jnp.where(l0 >= l2, 0, 2),
                     jnp.where(l1 >= l2, 1, 2)).astype(jnp.int32)
    pred_ref[...] = pred
    ph = lax.dot_general(x, wm_ref[...], (((1,), (1,)), ((), ())),
                         preferred_element_type=jnp.float32)
    laneD = lax.broadcasted_iota(jnp.int32, (TB, D), 1)
    ph_ref[...] = ph + jnp.where(laneD == 300, 1.0, 0.0).astype(jnp.float32)


@jax.jit
def _dense(x, W1, b1b, W2p, b2b, Wm_pad):
    return pl.pallas_call(
        _dense_body,
        grid=(NT,),
        in_specs=[
            pl.BlockSpec((TB, H), lambda i: (i, 0)),
            pl.BlockSpec((H, H), lambda i: (0, 0)),
            pl.BlockSpec((8, H), lambda i: (0, 0)),
            pl.BlockSpec((8, H), lambda i: (0, 0)),
            pl.BlockSpec((8, 8), lambda i: (0, 0)),
            pl.BlockSpec((D, H), lambda i: (0, 0)),
        ],
        out_specs=[
            pl.BlockSpec((TB, 3), lambda i: (i, 0)),
            pl.BlockSpec((TB, 1), lambda i: (i, 0)),
            pl.BlockSpec((TB, D), lambda i: (i, 0)),
        ],
        out_shape=[
            jax.ShapeDtypeStruct((M, 3), jnp.float32),
            jax.ShapeDtypeStruct((M, 1), jnp.int32),
            jax.ShapeDtypeStruct((M, D), jnp.float32),
        ],
    )(x, W1, b1b, W2p, b2b, Wm_pad)


def _pdt_body(wd_ref, ent_ref, out_ref):
    out_ref[...] = lax.dot_general(
        wd_ref[...], ent_ref[...], (((1,), (1,)), ((), ())),
        preferred_element_type=jnp.float32)


@jax.jit
def _pdt(Wd_pad, ent):
    return pl.pallas_call(
        _pdt_body,
        out_shape=jax.ShapeDtypeStruct((D, K), jnp.float32),
    )(Wd_pad, ent)


def _cumsum_body(ph_ref, v_ref, out_ref, carry_ref):
    @pl.when(pl.program_id(0) == 0)
    def _():
        carry_ref[...] = jnp.zeros_like(carry_ref)

    z = ph_ref[...] * v_ref[...]
    row = lax.broadcasted_iota(jnp.int32, (TB, TB), 0)
    col = lax.broadcasted_iota(jnp.int32, (TB, TB), 1)
    tri = (col <= row).astype(jnp.float32)
    p = lax.dot_general(tri, z, (((1,), (0,)), ((), ())),
                        preferred_element_type=jnp.float32)
    p = p + carry_ref[0:1, :]
    out_ref[...] = p
    carry_ref[0:1, :] = p[TB - 1:TB, :]


@jax.jit
def _cumsum(ph, vcol):
    return pl.pallas_call(
        _cumsum_body,
        grid=(NT,),
        in_specs=[
            pl.BlockSpec((TB, D), lambda i: (i, 0)),
            pl.BlockSpec((TB, 1), lambda i: (i, 0)),
        ],
        out_specs=pl.BlockSpec((TB, D), lambda i: (i, 0)),
        out_shape=jax.ShapeDtypeStruct((M, D), jnp.float32),
        scratch_shapes=[pltpu.VMEM((8, D), jnp.float32)],
    )(ph, vcol)


_NIDX = 2 * M
_CHUNK = 128


def _gather_body(p_hbm, idx_hbm, out_hbm, idx_v, rows_v, sem):
    nc = 2
    wid = lax.axis_index("s") * nc + lax.axis_index("c")
    per_w = _NIDX // 32

    def chunk(j, c):
        base = pl.multiple_of(wid * per_w + j * _CHUNK, _CHUNK)
        pltpu.sync_copy(idx_hbm.at[pl.ds(base, _CHUNK)], idx_v)
        pltpu.async_copy(p_hbm.at[idx_v], rows_v, sem).wait()
        pltpu.sync_copy(rows_v, out_hbm.at[pl.ds(base, _CHUNK)])
        return c

    lax.fori_loop(0, per_w // _CHUNK, chunk, 0)


@jax.jit
def _gather(P, idx_all):
    mesh = plsc.VectorSubcoreMesh(core_axis_name="c", subcore_axis_name="s")
    f = pl.kernel(
        _gather_body,
        mesh=mesh,
        out_type=jax.ShapeDtypeStruct((_NIDX, D), jnp.float32),
        scratch_types=[
            pltpu.VMEM((_CHUNK,), jnp.int32),
            pltpu.VMEM((_CHUNK, D), jnp.float32),
            pltpu.SemaphoreType.DMA,
        ],
    )
    return f(P, idx_all)


def _score_body(hi_ref, lo_ref, whi_ref, wlo_ref, pdt_ref, out_ref):
    sums = hi_ref[...] * whi_ref[...] + lo_ref[...] * wlo_ref[...]
    cnt = sums[:, 300:301]
    denom = jnp.maximum(cnt, 1.0)
    scores = lax.dot_general(sums, pdt_ref[...], (((1,), (0,)), ((), ())),
                             preferred_element_type=jnp.float32) / denom
    m = jnp.max(scores, axis=1, keepdims=True)
    lse = m + jnp.log(jnp.sum(jnp.exp(scores - m), axis=1, keepdims=True))
    out_ref[...] = scores - lse


@jax.jit
def _score(g, whi, wlo, pdT):
    return pl.pallas_call(
        _score_body,
        grid=(NT,),
        in_specs=[
            pl.BlockSpec((TB, D), lambda i: (i, 0)),
            pl.BlockSpec((TB, D), lambda i: (i + NT, 0)),
            pl.BlockSpec((TB, 1), lambda i: (i, 0)),
            pl.BlockSpec((TB, 1), lambda i: (i, 0)),
            pl.BlockSpec((D, K), lambda i: (0, 0)),
        ],
        out_specs=pl.BlockSpec((TB, K), lambda i: (i, 0)),
        out_shape=jax.ShapeDtypeStruct((M, K), jnp.float32),
    )(g, g, whi, wlo, pdT)


def _indices_for(flat_labels):
    valid = flat_labels != 0
    is_one = flat_labels == 1
    rho = jnp.cumsum(valid.astype(jnp.int32)) - 1
    big = jnp.int32(3 * M)
    one_rank = jnp.where(is_one, rho, big)
    idx_sorted = jnp.sort(one_rank)
    idx0 = idx_sorted[0]
    seg = jnp.searchsorted(idx_sorted[1:], rho + idx0, side='right')
    seg = jnp.clip(seg, 0, M - 1).astype(jnp.int32)
    svals = jnp.arange(M, dtype=jnp.int32)
    ends = jnp.searchsorted(seg, svals, side='right').astype(jnp.int32)
    starts = jnp.searchsorted(seg, svals, side='left').astype(jnp.int32)
    labels2d = flat_labels.reshape(B, L)
    count_ones = (labels2d == 1).sum(axis=1).astype(jnp.int32)
    off = jnp.cumsum(count_ones) - count_ones
    pos = jnp.arange(L, dtype=jnp.int32)
    s_id = off[:, None] + pos[None, :]
    maskv = pos[None, :] < count_ones[:, None]
    s_idc = jnp.clip(s_id, 0, M - 1).reshape(-1)
    lo = starts[s_idc]
    hi = ends[s_idc]
    mf = maskv.reshape(-1)
    ihi = jnp.maximum(hi - 1, 0)
    ilo = jnp.maximum(lo - 1, 0)
    whi = jnp.where(mf, 1.0, 0.0).astype(jnp.float32)
    wlo = jnp.where(mf & (lo > 0), -1.0, 0.0).astype(jnp.float32)
    idx_all = jnp.concatenate([ihi, ilo]).astype(jnp.int32)
    vcol = valid.astype(jnp.float32)[:, None]
    return vcol, idx_all, whi[:, None], wlo[:, None]


@jax.jit
def kernel(bio_slot_labels, hidden_states, entity_type_embeddings,
           W1, b1, W2, b2, Wm, Wd):
    x = hidden_states.reshape(M, H)
    b1b = jnp.broadcast_to(b1[None, :], (8, H))
    W2p = jnp.zeros((8, H), jnp.float32).at[0:3].set(W2)
    b2b = jnp.zeros((8, 8), jnp.float32).at[0, 0:3].set(b2)
    Wm_pad = jnp.zeros((D, H), jnp.float32).at[0:300].set(Wm)
    Wd_pad = jnp.zeros((D, H), jnp.float32).at[0:300].set(Wd)

    out3, pred, ph = _dense(x, W1, b1b, W2p, b2b, Wm_pad)
    pdT = _pdt(Wd_pad, entity_type_embeddings)

    def pool_score(flat_lab):
        vcol, idx_all, whi, wlo = _indices_for(flat_lab)
        P = _cumsum(ph, vcol)
        g = _gather(P, idx_all)
        return _score(g, whi, wlo, pdT)

    s1 = pool_score(bio_slot_labels.reshape(-1))
    s2 = pool_score(pred.reshape(-1))
    return (out3.reshape(B, L, 3), s1.reshape(B, L, K), s2.reshape(B, L, K))


# speedup vs baseline: 5.7283x; 5.7283x over previous
import functools
import jax
import jax.numpy as jnp
from jax import lax
from jax.experimental import pallas as pl
from jax.experimental.pallas import tpu as pltpu
from jax.experimental.pallas import tpu_sc as plsc

B, L, H, K = 8, 2048, 768, 256
M = B * L
D = 384
TB = 512
NT = M // TB


def _dense_body(x_ref, w1_ref, b1_ref, w2_ref, b2_ref, wm_ref,
                out3_ref, pred_ref, ph_ref):
    x = x_ref[...]
    h1 = lax.dot_general(x, w1_ref[...], (((1,), (1,)), ((), ())),
                         preferred_element_type=jnp.float32)
    h1 = jnp.maximum(h1 + b1_ref[0:1, :], 0.0)
    logits8 = lax.dot_general(h1, w2_ref[...], (((1,), (1,)), ((), ())),
                              preferred_element_type=jnp.float32)
    logits8 = logits8 + b2_ref[0:1, :]
    lane8 = lax.broadcasted_iota(jnp.int32, (TB, 8), 1)
    neg = jnp.where(lane8 >= 3, -1e30, 0.0).astype(jnp.float32)
    lm = logits8 + neg
    m = jnp.max(lm, axis=1, keepdims=True)
    lse = m + jnp.log(jnp.sum(jnp.exp(lm - m), axis=1, keepdims=True))
    out8 = lm - lse
    out3_ref[...] = out8[:, 0:3]
    l0 = logits8[:, 0:1]
    l1 = logits8[:, 1:2]
    l2 = logits8[:, 2:3]
    pred = jnp.where(l0 >= l1,
                     jnp.where(l0 >= l2, 0, 2),
                     jnp.where(l1 >= l2, 1, 2)).astype(jnp.int32)
    pred_ref[...] = pred
    ph = lax.dot_general(x, wm_ref[...], (((1,), (1,)), ((), ())),
                         preferred_element_type=jnp.float32)
    laneD = lax.broadcasted_iota(jnp.int32, (TB, D), 1)
    ph_ref[...] = ph + jnp.where(laneD == 300, 1.0, 0.0).astype(jnp.float32)


@jax.jit
def _dense(x, W1, b1b, W2p, b2b, Wm_pad):
    return pl.pallas_call(
        _dense_body,
        grid=(NT,),
        in_specs=[
            pl.BlockSpec((TB, H), lambda i: (i, 0)),
            pl.BlockSpec((H, H), lambda i: (0, 0)),
            pl.BlockSpec((8, H), lambda i: (0, 0)),
            pl.BlockSpec((8, H), lambda i: (0, 0)),
            pl.BlockSpec((8, 8), lambda i: (0, 0)),
            pl.BlockSpec((D, H), lambda i: (0, 0)),
        ],
        out_specs=[
            pl.BlockSpec((TB, 3), lambda i: (i, 0)),
            pl.BlockSpec((TB, 1), lambda i: (i, 0)),
            pl.BlockSpec((TB, D), lambda i: (i, 0)),
        ],
        out_shape=[
            jax.ShapeDtypeStruct((M, 3), jnp.float32),
            jax.ShapeDtypeStruct((M, 1), jnp.int32),
            jax.ShapeDtypeStruct((M, D), jnp.float32),
        ],
    )(x, W1, b1b, W2p, b2b, Wm_pad)


def _pdt_body(wd_ref, ent_ref, out_ref):
    out_ref[...] = lax.dot_general(
        wd_ref[...], ent_ref[...], (((1,), (1,)), ((), ())),
        preferred_element_type=jnp.float32)


@jax.jit
def _pdt(Wd_pad, ent):
    return pl.pallas_call(
        _pdt_body,
        out_shape=jax.ShapeDtypeStruct((D, K), jnp.float32),
    )(Wd_pad, ent)


def _cumsum_body(ph_ref, v_ref, out_ref, carry_ref):
    @pl.when(pl.program_id(0) == 0)
    def _():
        carry_ref[...] = jnp.zeros_like(carry_ref)

    z = ph_ref[...] * v_ref[...]
    row = lax.broadcasted_iota(jnp.int32, (TB, TB), 0)
    col = lax.broadcasted_iota(jnp.int32, (TB, TB), 1)
    tri = (col <= row).astype(jnp.float32)
    p = lax.dot_general(tri, z, (((1,), (0,)), ((), ())),
                        preferred_element_type=jnp.float32)
    p = p + carry_ref[0:1, :]
    out_ref[...] = p
    carry_ref[0:1, :] = p[TB - 1:TB, :]


@jax.jit
def _cumsum(ph, vcol):
    return pl.pallas_call(
        _cumsum_body,
        grid=(NT,),
        in_specs=[
            pl.BlockSpec((TB, D), lambda i: (i, 0)),
            pl.BlockSpec((TB, 1), lambda i: (i, 0)),
        ],
        out_specs=pl.BlockSpec((TB, D), lambda i: (i, 0)),
        out_shape=jax.ShapeDtypeStruct((M, D), jnp.float32),
        scratch_shapes=[pltpu.VMEM((8, D), jnp.float32)],
    )(ph, vcol)


_CHUNK = 128
_PW = M // 32


def _gather_body(p_hbm, st_hbm, en_hbm, sid_hbm, wh_hbm,
                 out_hbm, wl_hbm,
                 sid_v, whv, lov, hiv, idxhi, idxlo, wlv, rows_v, sem):
    nc = 2
    w = lax.axis_index("s") * nc + lax.axis_index("c")
    base = pl.multiple_of(w * _PW, _PW)
    pltpu.sync_copy(sid_hbm.at[pl.ds(base, _PW)], sid_v)
    pltpu.sync_copy(wh_hbm.at[pl.ds(base, _PW)], whv)
    for c in range(_PW // _CHUNK):
        sc_ref = sid_v.at[pl.ds(c * _CHUNK, _CHUNK)]
        pltpu.async_copy(st_hbm.at[sc_ref], lov, sem).wait()
        pltpu.async_copy(en_hbm.at[sc_ref], hiv, sem).wait()
        for v in range(_CHUNK // 16):
            sl = pl.ds(v * 16, 16)
            gl = pl.ds(c * _CHUNK + v * 16, 16)
            lo = lov[sl]
            hi = hiv[sl]
            whi16 = whv[gl]
            idxhi[sl] = jnp.maximum(hi - 1, 0)
            idxlo[sl] = jnp.maximum(lo - 1, 0)
            wlv[gl] = jnp.where((whi16 > 0.0) & (lo > 0), -1.0, 0.0
                                ).astype(jnp.float32)
        cb = pl.multiple_of(base + c * _CHUNK, _CHUNK)
        pltpu.async_copy(p_hbm.at[idxhi], rows_v, sem).wait()
        pltpu.sync_copy(rows_v, out_hbm.at[pl.ds(cb, _CHUNK)])
        pltpu.async_copy(p_hbm.at[idxlo], rows_v, sem).wait()
        pltpu.sync_copy(rows_v, out_hbm.at[pl.ds(M + cb, _CHUNK)])
    pltpu.sync_copy(wlv, wl_hbm.at[pl.ds(base, _PW)])


@jax.jit
def _gather(P, starts, ends, sid, whi):
    mesh = plsc.VectorSubcoreMesh(core_axis_name="c", subcore_axis_name="s")
    f = pl.kernel(
        _gather_body,
        mesh=mesh,
        out_type=[
            jax.ShapeDtypeStruct((2 * M, D), jnp.float32),
            jax.ShapeDtypeStruct((M,), jnp.float32),
        ],
        scratch_types=[
            pltpu.VMEM((_PW,), jnp.int32),
            pltpu.VMEM((_PW,), jnp.float32),
            pltpu.VMEM((_CHUNK,), jnp.int32),
            pltpu.VMEM((_CHUNK,), jnp.int32),
            pltpu.VMEM((_CHUNK,), jnp.int32),
            pltpu.VMEM((_CHUNK,), jnp.int32),
            pltpu.VMEM((_PW,), jnp.float32),
            pltpu.VMEM((_CHUNK, D), jnp.float32),
            pltpu.SemaphoreType.DMA,
        ],
    )
    return f(P, starts, ends, sid, whi)


def _score_body(hi_ref, lo_ref, whi_ref, wlo_ref, pdt_ref, out_ref):
    sums = hi_ref[...] * whi_ref[...] + lo_ref[...] * wlo_ref[...]
    cnt = sums[:, 300:301]
    denom = jnp.maximum(cnt, 1.0)
    scores = lax.dot_general(sums, pdt_ref[...], (((1,), (0,)), ((), ())),
                             preferred_element_type=jnp.float32) / denom
    m = jnp.max(scores, axis=1, keepdims=True)
    lse = m + jnp.log(jnp.sum(jnp.exp(scores - m), axis=1, keepdims=True))
    out_ref[...] = scores - lse


@jax.jit
def _score(g, whi, wlo, pdT):
    return pl.pallas_call(
        _score_body,
        grid=(NT,),
        in_specs=[
            pl.BlockSpec((TB, D), lambda i: (i, 0)),
            pl.BlockSpec((TB, D), lambda i: (i + NT, 0)),
            pl.BlockSpec((TB, 1), lambda i: (i, 0)),
            pl.BlockSpec((TB, 1), lambda i: (i, 0)),
            pl.BlockSpec((D, K), lambda i: (0, 0)),
        ],
        out_specs=pl.BlockSpec((TB, K), lambda i: (i, 0)),
        out_shape=jax.ShapeDtypeStruct((M, K), jnp.float32),
    )(g, g, whi, wlo, pdT)


def _indices_for(flat_labels):
    valid = flat_labels != 0
    is_one = flat_labels == 1
    rho = jnp.cumsum(valid.astype(jnp.int32)) - 1
    big = jnp.int32(3 * M)
    r0 = jnp.min(jnp.where(is_one, rho, big))
    obr = jnp.zeros((M,), jnp.int32).at[
        jnp.where(is_one, rho, M)].add(1, mode='drop')
    G = jnp.cumsum(obr)
    t = jnp.clip(rho + r0, 0, M - 1)
    Gt = jnp.where(rho + r0 >= 0, G[t], 0)
    seg = jnp.clip(Gt - (rho >= 0).astype(jnp.int32), 0, M - 1)
    cnt_at = jnp.zeros((M,), jnp.int32).at[seg].add(1)
    ends = jnp.cumsum(cnt_at).astype(jnp.int32)
    starts = (ends - cnt_at).astype(jnp.int32)
    labels2d = flat_labels.reshape(B, L)
    count_ones = (labels2d == 1).sum(axis=1).astype(jnp.int32)
    off = jnp.cumsum(count_ones) - count_ones
    pos = jnp.arange(L, dtype=jnp.int32)
    sid = jnp.minimum(off[:, None] + pos[None, :], M - 1).reshape(-1)
    whi = (pos[None, :] < count_ones[:, None]).astype(jnp.float32).reshape(-1)
    vcol = valid.astype(jnp.float32)[:, None]
    return vcol, starts, ends, sid, whi


@jax.jit
def kernel(bio_slot_labels, hidden_states, entity_type_embeddings,
           W1, b1, W2, b2, Wm, Wd):
    x = hidden_states.reshape(M, H)
    b1b = jnp.broadcast_to(b1[None, :], (8, H))
    W2p = jnp.zeros((8, H), jnp.float32).at[0:3].set(W2)
    b2b = jnp.zeros((8, 8), jnp.float32).at[0, 0:3].set(b2)
    Wm_pad = jnp.zeros((D, H), jnp.float32).at[0:300].set(Wm)
    Wd_pad = jnp.zeros((D, H), jnp.float32).at[0:300].set(Wd)

    out3, pred, ph = _dense(x, W1, b1b, W2p, b2b, Wm_pad)
    pdT = _pdt(Wd_pad, entity_type_embeddings)

    def pool_score(flat_lab):
        vcol, starts, ends, sid, whi = _indices_for(flat_lab)
        P = _cumsum(ph, vcol)
        g, wl = _gather(P, starts, ends, sid, whi)
        return _score(g, whi[:, None], wl[:, None], pdT)

    s1 = pool_score(bio_slot_labels.reshape(-1))
    s2 = pool_score(pred.reshape(-1))
    return (out3.reshape(B, L, 3), s1.reshape(B, L, K), s2.reshape(B, L, K))


# speedup vs baseline: 5.8136x; 1.0149x over previous
import functools
import jax
import jax.numpy as jnp
from jax import lax
from jax.experimental import pallas as pl
from jax.experimental.pallas import tpu as pltpu
from jax.experimental.pallas import tpu_sc as plsc

B, L, H, K = 8, 2048, 768, 256
M = B * L
D = 384
TB = 512
NT = M // TB


def _dense_body(x_ref, w1_ref, b1_ref, w2_ref, b2_ref, wm_ref,
                out3_ref, pred_ref, ph_ref):
    x = x_ref[...]
    h1 = lax.dot_general(x, w1_ref[...], (((1,), (1,)), ((), ())),
                         preferred_element_type=jnp.float32)
    h1 = jnp.maximum(h1 + b1_ref[0:1, :], 0.0)
    logits8 = lax.dot_general(h1, w2_ref[...], (((1,), (1,)), ((), ())),
                              preferred_element_type=jnp.float32)
    logits8 = logits8 + b2_ref[0:1, :]
    lane8 = lax.broadcasted_iota(jnp.int32, (TB, 8), 1)
    neg = jnp.where(lane8 >= 3, -1e30, 0.0).astype(jnp.float32)
    lm = logits8 + neg
    m = jnp.max(lm, axis=1, keepdims=True)
    lse = m + jnp.log(jnp.sum(jnp.exp(lm - m), axis=1, keepdims=True))
    out8 = lm - lse
    out3_ref[...] = out8[:, 0:3]
    l0 = logits8[:, 0:1]
    l1 = logits8[:, 1:2]
    l2 = logits8[:, 2:3]
    pred = jnp.where(l0 >= l1,
                     jnp.where(l0 >= l2, 0, 2),
                     jnp.where(l1 >= l2, 1, 2)).astype(jnp.int32)
    pred_ref[...] = pred
    ph = lax.dot_general(x, wm_ref[...], (((1,), (1,)), ((), ())),
                         preferred_element_type=jnp.float32)
    laneD = lax.broadcasted_iota(jnp.int32, (TB, D), 1)
    ph_ref[...] = ph + jnp.where(laneD == 300, 1.0, 0.0).astype(jnp.float32)


@jax.jit
def _dense(x, W1, b1b, W2p, b2b, Wm_pad):
    return pl.pallas_call(
        _dense_body,
        grid=(NT,),
        in_specs=[
            pl.BlockSpec((TB, H), lambda i: (i, 0)),
            pl.BlockSpec((H, H), lambda i: (0, 0)),
            pl.BlockSpec((8, H), lambda i: (0, 0)),
            pl.BlockSpec((8, H), lambda i: (0, 0)),
            pl.BlockSpec((8, 8), lambda i: (0, 0)),
            pl.BlockSpec((D, H), lambda i: (0, 0)),
        ],
        out_specs=[
            pl.BlockSpec((TB, 3), lambda i: (i, 0)),
            pl.BlockSpec((TB, 1), lambda i: (i, 0)),
            pl.BlockSpec((TB, D), lambda i: (i, 0)),
        ],
        out_shape=[
            jax.ShapeDtypeStruct((M, 3), jnp.float32),
            jax.ShapeDtypeStruct((M, 1), jnp.int32),
            jax.ShapeDtypeStruct((M, D), jnp.float32),
        ],
    )(x, W1, b1b, W2p, b2b, Wm_pad)


def _pdt_body(wd_ref, ent_ref, out_ref):
    out_ref[...] = lax.dot_general(
        wd_ref[...], ent_ref[...], (((1,), (1,)), ((), ())),
        preferred_element_type=jnp.float32)


@jax.jit
def _pdt(Wd_pad, ent):
    return pl.pallas_call(
        _pdt_body,
        out_shape=jax.ShapeDtypeStruct((D, K), jnp.float32),
    )(Wd_pad, ent)


def _cumsum_body(ph_ref, v_ref, out_ref, carry_ref):
    @pl.when(pl.program_id(0) == 0)
    def _():
        carry_ref[...] = jnp.zeros_like(carry_ref)

    z = ph_ref[...] * v_ref[...]
    row = lax.broadcasted_iota(jnp.int32, (TB, TB), 0)
    col = lax.broadcasted_iota(jnp.int32, (TB, TB), 1)
    tri = (col <= row).astype(jnp.float32)
    p = lax.dot_general(tri, z, (((1,), (0,)), ((), ())),
                        preferred_element_type=jnp.float32)
    p = p + carry_ref[0:1, :]
    out_ref[...] = p
    carry_ref[0:1, :] = p[TB - 1:TB, :]


@jax.jit
def _cumsum(ph, vcol):
    return pl.pallas_call(
        _cumsum_body,
        grid=(NT,),
        in_specs=[
            pl.BlockSpec((TB, D), lambda i: (i, 0)),
            pl.BlockSpec((TB, 1), lambda i: (i, 0)),
        ],
        out_specs=pl.BlockSpec((TB, D), lambda i: (i, 0)),
        out_shape=jax.ShapeDtypeStruct((M, D), jnp.float32),
        scratch_shapes=[pltpu.VMEM((8, D), jnp.float32)],
    )(ph, vcol)


_CHUNK = 128
_PW = M // 32


def _gather_body(p_hbm, st_hbm, en_hbm, sid_hbm, wh_hbm,
                 out_hbm, wl_hbm,
                 sid_v, whv, lov, hiv, idxhi, idxlo, wlv, rowA, rowB,
                 semG, semW):
    nc = 2
    w = lax.axis_index("s") * nc + lax.axis_index("c")
    base = pl.multiple_of(w * _PW, _PW)
    pltpu.sync_copy(sid_hbm.at[pl.ds(base, _PW)], sid_v)
    pltpu.sync_copy(wh_hbm.at[pl.ds(base, _PW)], whv)
    lk = []
    for c in range(_PW // _CHUNK):
        cs = pl.ds(c * _CHUNK, _CHUNK)
        sc_ref = sid_v.at[cs]
        lk.append(pltpu.async_copy(st_hbm.at[sc_ref], lov.at[cs], semG))
        lk.append(pltpu.async_copy(en_hbm.at[sc_ref], hiv.at[cs], semG))
    for h in lk:
        h.wait()
    for v in range(_PW // 16):
        sl = pl.ds(v * 16, 16)
        lo = lov[sl]
        hi = hiv[sl]
        whi16 = whv[sl]
        idxhi[sl] = jnp.maximum(hi - 1, 0)
        idxlo[sl] = jnp.maximum(lo - 1, 0)
        wlv[sl] = jnp.where((whi16 > 0.0) & (lo > 0), -1.0, 0.0
                            ).astype(jnp.float32)
    bufs = [rowA, rowB]
    wrs = []
    for k in range(2 * (_PW // _CHUNK)):
        c = k // 2
        src_idx = (idxhi if k % 2 == 0 else idxlo).at[pl.ds(c * _CHUNK, _CHUNK)]
        half = 0 if k % 2 == 0 else M
        buf = bufs[k % 2]
        if k >= 2:
            wrs[k - 2].wait()
        pltpu.async_copy(p_hbm.at[src_idx], buf, semG).wait()
        dst = out_hbm.at[pl.ds(pl.multiple_of(half + base + c * _CHUNK,
                                              _CHUNK), _CHUNK)]
        wrs.append(pltpu.async_copy(buf, dst, semW))
    wrs[-2].wait()
    wrs[-1].wait()
    pltpu.sync_copy(wlv, wl_hbm.at[pl.ds(base, _PW)])


@jax.jit
def _gather(P, starts, ends, sid, whi):
    mesh = plsc.VectorSubcoreMesh(core_axis_name="c", subcore_axis_name="s")
    f = pl.kernel(
        _gather_body,
        mesh=mesh,
        out_type=[
            jax.ShapeDtypeStruct((2 * M, D), jnp.float32),
            jax.ShapeDtypeStruct((M,), jnp.float32),
        ],
        scratch_types=[
            pltpu.VMEM((_PW,), jnp.int32),
            pltpu.VMEM((_PW,), jnp.float32),
            pltpu.VMEM((_PW,), jnp.int32),
            pltpu.VMEM((_PW,), jnp.int32),
            pltpu.VMEM((_PW,), jnp.int32),
            pltpu.VMEM((_PW,), jnp.int32),
            pltpu.VMEM((_PW,), jnp.float32),
            pltpu.VMEM((_CHUNK, D), jnp.float32),
            pltpu.VMEM((_CHUNK, D), jnp.float32),
            pltpu.SemaphoreType.DMA,
            pltpu.SemaphoreType.DMA,
        ],
    )
    return f(P, starts, ends, sid, whi)


def _score_body(hi_ref, lo_ref, whi_ref, wlo_ref, pdt_ref, out_ref):
    sums = hi_ref[...] * whi_ref[...] + lo_ref[...] * wlo_ref[...]
    cnt = sums[:, 300:301]
    denom = jnp.maximum(cnt, 1.0)
    scores = lax.dot_general(sums, pdt_ref[...], (((1,), (0,)), ((), ())),
                             preferred_element_type=jnp.float32) / denom
    m = jnp.max(scores, axis=1, keepdims=True)
    lse = m + jnp.log(jnp.sum(jnp.exp(scores - m), axis=1, keepdims=True))
    out_ref[...] = scores - lse


@jax.jit
def _score(g, whi, wlo, pdT):
    return pl.pallas_call(
        _score_body,
        grid=(NT,),
        in_specs=[
            pl.BlockSpec((TB, D), lambda i: (i, 0)),
            pl.BlockSpec((TB, D), lambda i: (i + NT, 0)),
            pl.BlockSpec((TB, 1), lambda i: (i, 0)),
            pl.BlockSpec((TB, 1), lambda i: (i, 0)),
            pl.BlockSpec((D, K), lambda i: (0, 0)),
        ],
        out_specs=pl.BlockSpec((TB, K), lambda i: (i, 0)),
        out_shape=jax.ShapeDtypeStruct((M, K), jnp.float32),
    )(g, g, whi, wlo, pdT)


def _indices_for(flat_labels):
    valid = flat_labels != 0
    is_one = flat_labels == 1
    rho = jnp.cumsum(valid.astype(jnp.int32)) - 1
    big = jnp.int32(3 * M)
    r0 = jnp.min(jnp.where(is_one, rho, big))
    obr = jnp.zeros((M,), jnp.int32).at[
        jnp.where(is_one, rho, M)].add(1, mode='drop')
    G = jnp.cumsum(obr)
    t = jnp.clip(rho + r0, 0, M - 1)
    Gt = jnp.where(rho + r0 >= 0, G[t], 0)
    seg = jnp.clip(Gt - (rho >= 0).astype(jnp.int32), 0, M - 1)
    cnt_at = jnp.zeros((M,), jnp.int32).at[seg].add(1)
    ends = jnp.cumsum(cnt_at).astype(jnp.int32)
    starts = (ends - cnt_at).astype(jnp.int32)
    labels2d = flat_labels.reshape(B, L)
    count_ones = (labels2d == 1).sum(axis=1).astype(jnp.int32)
    off = jnp.cumsum(count_ones) - count_ones
    pos = jnp.arange(L, dtype=jnp.int32)
    sid = jnp.minimum(off[:, None] + pos[None, :], M - 1).reshape(-1)
    whi = (pos[None, :] < count_ones[:, None]).astype(jnp.float32).reshape(-1)
    vcol = valid.astype(jnp.float32)[:, None]
    return vcol, starts, ends, sid, whi


@jax.jit
def kernel(bio_slot_labels, hidden_states, entity_type_embeddings,
           W1, b1, W2, b2, Wm, Wd):
    x = hidden_states.reshape(M, H)
    b1b = jnp.broadcast_to(b1[None, :], (8, H))
    W2p = jnp.zeros((8, H), jnp.float32).at[0:3].set(W2)
    b2b = jnp.zeros((8, 8), jnp.float32).at[0, 0:3].set(b2)
    Wm_pad = jnp.zeros((D, H), jnp.float32).at[0:300].set(Wm)
    Wd_pad = jnp.zeros((D, H), jnp.float32).at[0:300].set(Wd)

    out3, pred, ph = _dense(x, W1, b1b, W2p, b2b, Wm_pad)
    pdT = _pdt(Wd_pad, entity_type_embeddings)

    def pool_score(flat_lab):
        vcol, starts, ends, sid, whi = _indices_for(flat_lab)
        P = _cumsum(ph, vcol)
        g, wl = _gather(P, starts, ends, sid, whi)
        return _score(g, whi[:, None], wl[:, None], pdT)

    s1 = pool_score(bio_slot_labels.reshape(-1))
    s2 = pool_score(pred.reshape(-1))
    return (out3.reshape(B, L, 3), s1.reshape(B, L, K), s2.reshape(B, L, K))


# speedup vs baseline: 5.8329x; 1.0033x over previous
import jax
import jax.numpy as jnp
from jax import lax
from jax.experimental import pallas as pl
from jax.experimental.pallas import tpu as pltpu
from jax.experimental.pallas import tpu_sc as plsc

B, L, H, K = 8, 2048, 768, 256
M = B * L
D = 384
TB = 512
NT = M // TB


def _dense_body(x_ref, w1_ref, b1_ref, w2_ref, b2_ref, wm_ref,
                out3_ref, pred_ref, ph_ref):
    x = x_ref[...]
    h1 = lax.dot_general(x, w1_ref[...], (((1,), (1,)), ((), ())),
                         preferred_element_type=jnp.float32)
    h1 = jnp.maximum(h1 + b1_ref[0:1, :], 0.0)
    logits8 = lax.dot_general(h1, w2_ref[...], (((1,), (1,)), ((), ())),
                              preferred_element_type=jnp.float32)
    logits8 = logits8 + b2_ref[0:1, :]
    lane8 = lax.broadcasted_iota(jnp.int32, (TB, 8), 1)
    neg = jnp.where(lane8 >= 3, -1e30, 0.0).astype(jnp.float32)
    lm = logits8 + neg
    m = jnp.max(lm, axis=1, keepdims=True)
    lse = m + jnp.log(jnp.sum(jnp.exp(lm - m), axis=1, keepdims=True))
    out8 = lm - lse
    out3_ref[...] = out8[:, 0:3]
    l0 = logits8[:, 0:1]
    l1 = logits8[:, 1:2]
    l2 = logits8[:, 2:3]
    pred = jnp.where(l0 >= l1,
                     jnp.where(l0 >= l2, 0, 2),
                     jnp.where(l1 >= l2, 1, 2)).astype(jnp.int32)
    pred_ref[...] = pred
    ph = lax.dot_general(x, wm_ref[...], (((1,), (1,)), ((), ())),
                         preferred_element_type=jnp.float32)
    laneD = lax.broadcasted_iota(jnp.int32, (TB, D), 1)
    ph_ref[...] = ph + jnp.where(laneD == 300, 1.0, 0.0).astype(jnp.float32)


@jax.jit
def _dense(x, W1, b1b, W2p, b2b, Wm_pad):
    return pl.pallas_call(
        _dense_body,
        grid=(NT,),
        in_specs=[
            pl.BlockSpec((TB, H), lambda i: (i, 0)),
            pl.BlockSpec((H, H), lambda i: (0, 0)),
            pl.BlockSpec((8, H), lambda i: (0, 0)),
            pl.BlockSpec((8, H), lambda i: (0, 0)),
            pl.BlockSpec((8, 8), lambda i: (0, 0)),
            pl.BlockSpec((D, H), lambda i: (0, 0)),
        ],
        out_specs=[
            pl.BlockSpec((TB, 3), lambda i: (i, 0)),
            pl.BlockSpec((TB, 1), lambda i: (i, 0)),
            pl.BlockSpec((TB, D), lambda i: (i, 0)),
        ],
        out_shape=[
            jax.ShapeDtypeStruct((M, 3), jnp.float32),
            jax.ShapeDtypeStruct((M, 1), jnp.int32),
            jax.ShapeDtypeStruct((M, D), jnp.float32),
        ],
    )(x, W1, b1b, W2p, b2b, Wm_pad)


def _pdt_body(wd_ref, ent_ref, out_ref):
    out_ref[...] = lax.dot_general(
        wd_ref[...], ent_ref[...], (((1,), (1,)), ((), ())),
        preferred_element_type=jnp.float32)


@jax.jit
def _pdt(Wd_pad, ent):
    return pl.pallas_call(
        _pdt_body,
        out_shape=jax.ShapeDtypeStruct((D, K), jnp.float32),
    )(Wd_pad, ent)


def _cumsum_body(ph_ref, v_ref, out_ref, carry_ref):
    @pl.when(pl.program_id(0) == 0)
    def _():
        carry_ref[...] = jnp.zeros_like(carry_ref)

    z = ph_ref[...] * v_ref[...]
    row = lax.broadcasted_iota(jnp.int32, (TB, TB), 0)
    col = lax.broadcasted_iota(jnp.int32, (TB, TB), 1)
    tri = (col <= row).astype(jnp.float32)
    p = lax.dot_general(tri, z, (((1,), (0,)), ((), ())),
                        preferred_element_type=jnp.float32)
    p = p + carry_ref[0:1, :]
    out_ref[...] = p
    carry_ref[0:1, :] = p[TB - 1:TB, :]


@jax.jit
def _cumsum(ph, vcol):
    return pl.pallas_call(
        _cumsum_body,
        grid=(NT,),
        in_specs=[
            pl.BlockSpec((TB, D), lambda i: (i, 0)),
            pl.BlockSpec((TB, 1), lambda i: (i, 0)),
        ],
        out_specs=pl.BlockSpec((TB, D), lambda i: (i, 0)),
        out_shape=jax.ShapeDtypeStruct((M, D), jnp.float32),
        scratch_shapes=[pltpu.VMEM((8, D), jnp.float32)],
    )(ph, vcol)


_CHUNK = 128
_PW = M // 32


def _gather_body(p_hbm, st_hbm, en_hbm, sid_hbm, wh_hbm,
                 out_hbm, wl_hbm,
                 sid_v, whv, lov, hiv, idxhi, idxlo, wlv, rowA, rowB,
                 semG, semW):
    nc = 2
    w = lax.axis_index("s") * nc + lax.axis_index("c")
    base = pl.multiple_of(w * _PW, _PW)
    pltpu.sync_copy(sid_hbm.at[pl.ds(base, _PW)], sid_v)
    pltpu.sync_copy(wh_hbm.at[pl.ds(base, _PW)], whv)
    lk = []
    for c in range(_PW // _CHUNK):
        cs = pl.ds(c * _CHUNK, _CHUNK)
        sc_ref = sid_v.at[cs]
        lk.append(pltpu.async_copy(st_hbm.at[sc_ref], lov.at[cs], semG))
        lk.append(pltpu.async_copy(en_hbm.at[sc_ref], hiv.at[cs], semG))
    for h in lk:
        h.wait()
    for v in range(_PW // 16):
        sl = pl.ds(v * 16, 16)
        lo = lov[sl]
        hi = hiv[sl]
        whi16 = whv[sl]
        idxhi[sl] = jnp.maximum(hi - 1, 0)
        idxlo[sl] = jnp.maximum(lo - 1, 0)
        wlv[sl] = jnp.where((whi16 > 0.0) & (lo > 0), -1.0, 0.0
                            ).astype(jnp.float32)
    bufs = [rowA, rowB]
    wrs = []
    for k in range(2 * (_PW // _CHUNK)):
        c = k // 2
        src_idx = (idxhi if k % 2 == 0 else idxlo).at[pl.ds(c * _CHUNK, _CHUNK)]
        half = 0 if k % 2 == 0 else M
        buf = bufs[k % 2]
        if k >= 2:
            wrs[k - 2].wait()
        pltpu.async_copy(p_hbm.at[src_idx], buf, semG).wait()
        dst = out_hbm.at[pl.ds(pl.multiple_of(half + base + c * _CHUNK,
                                              _CHUNK), _CHUNK)]
        wrs.append(pltpu.async_copy(buf, dst, semW))
    wrs[-2].wait()
    wrs[-1].wait()
    pltpu.sync_copy(wlv, wl_hbm.at[pl.ds(base, _PW)])


@jax.jit
def _gather(P, starts, ends, sid, whi):
    mesh = plsc.VectorSubcoreMesh(core_axis_name="c", subcore_axis_name="s")
    f = pl.kernel(
        _gather_body,
        mesh=mesh,
        out_type=[
            jax.ShapeDtypeStruct((2 * M, D), jnp.float32),
            jax.ShapeDtypeStruct((M,), jnp.float32),
        ],
        scratch_types=[
            pltpu.VMEM((_PW,), jnp.int32),
            pltpu.VMEM((_PW,), jnp.float32),
            pltpu.VMEM((_PW,), jnp.int32),
            pltpu.VMEM((_PW,), jnp.int32),
            pltpu.VMEM((_PW,), jnp.int32),
            pltpu.VMEM((_PW,), jnp.int32),
            pltpu.VMEM((_PW,), jnp.float32),
            pltpu.VMEM((_CHUNK, D), jnp.float32),
            pltpu.VMEM((_CHUNK, D), jnp.float32),
            pltpu.SemaphoreType.DMA,
            pltpu.SemaphoreType.DMA,
        ],
    )
    return f(P, starts, ends, sid, whi)


def _score_body(hi_ref, lo_ref, whi_ref, wlo_ref, pdt_ref, out_ref):
    sums = hi_ref[...] * whi_ref[...] + lo_ref[...] * wlo_ref[...]
    cnt = sums[:, 300:301]
    denom = jnp.maximum(cnt, 1.0)
    scores = lax.dot_general(sums, pdt_ref[...], (((1,), (0,)), ((), ())),
                             preferred_element_type=jnp.float32) / denom
    m = jnp.max(scores, axis=1, keepdims=True)
    lse = m + jnp.log(jnp.sum(jnp.exp(scores - m), axis=1, keepdims=True))
    out_ref[...] = scores - lse


@jax.jit
def _score(g, whi, wlo, pdT):
    return pl.pallas_call(
        _score_body,
        grid=(NT,),
        in_specs=[
            pl.BlockSpec((TB, D), lambda i: (i, 0)),
            pl.BlockSpec((TB, D), lambda i: (i + NT, 0)),
            pl.BlockSpec((TB, 1), lambda i: (i, 0)),
            pl.BlockSpec((TB, 1), lambda i: (i, 0)),
            pl.BlockSpec((D, K), lambda i: (0, 0)),
        ],
        out_specs=pl.BlockSpec((TB, K), lambda i: (i, 0)),
        out_shape=jax.ShapeDtypeStruct((M, K), jnp.float32),
    )(g, g, whi, wlo, pdT)


def _indices_for(flat_labels):
    valid = flat_labels != 0
    is_one = flat_labels == 1
    rho = jnp.cumsum(valid.astype(jnp.int32)) - 1
    big = jnp.int32(3 * M)
    r0 = jnp.min(jnp.where(is_one, rho, big))
    obr = jnp.zeros((M,), jnp.int32).at[
        jnp.where(is_one, rho, M)].add(1, mode='drop')
    G = jnp.cumsum(obr)
    t = jnp.clip(rho + r0, 0, M - 1)
    Gt = jnp.where(rho + r0 >= 0, G[t], 0)
    seg = jnp.clip(Gt - (rho >= 0).astype(jnp.int32), 0, M - 1)
    cnt_at = jnp.zeros((M,), jnp.int32).at[seg].add(1)
    ends = jnp.cumsum(cnt_at).astype(jnp.int32)
    starts = (ends - cnt_at).astype(jnp.int32)
    labels2d = flat_labels.reshape(B, L)
    count_ones = (labels2d == 1).sum(axis=1).astype(jnp.int32)
    off = jnp.cumsum(count_ones) - count_ones
    pos = jnp.arange(L, dtype=jnp.int32)
    sid = jnp.minimum(off[:, None] + pos[None, :], M - 1).reshape(-1)
    whi = (pos[None, :] < count_ones[:, None]).astype(jnp.float32).reshape(-1)
    vcol = valid.astype(jnp.float32)[:, None]
    return vcol, starts, ends, sid, whi


@jax.jit
def kernel(bio_slot_labels, hidden_states, entity_type_embeddings,
           W1, b1, W2, b2, Wm, Wd):
    x = hidden_states.reshape(M, H)
    b1b = jnp.broadcast_to(b1[None, :], (8, H))
    W2p = jnp.zeros((8, H), jnp.float32).at[0:3].set(W2)
    b2b = jnp.zeros((8, 8), jnp.float32).at[0, 0:3].set(b2)
    Wm_pad = jnp.zeros((D, H), jnp.float32).at[0:300].set(Wm)
    Wd_pad = jnp.zeros((D, H), jnp.float32).at[0:300].set(Wd)

    out3, pred, ph = _dense(x, W1, b1b, W2p, b2b, Wm_pad)
    pdT = _pdt(Wd_pad, entity_type_embeddings)

    def pool_score(flat_lab):
        vcol, starts, ends, sid, whi = _indices_for(flat_lab)
        P = _cumsum(ph, vcol)
        g, wl = _gather(P, starts, ends, sid, whi)
        return _score(g, whi[:, None], wl[:, None], pdT)

    s1 = pool_score(bio_slot_labels.reshape(-1))
    s2 = pool_score(pred.reshape(-1))
    return (out3.reshape(B, L, 3), s1.reshape(B, L, K), s2.reshape(B, L, K))
